# baseline (speedup 1.0000x reference)
import jax
import jax.numpy as jnp
from jax import lax
from jax.experimental import pallas as pl
from jax.experimental.pallas import tpu as pltpu
from jax.experimental.pallas import tpu_sc as plsc

N_FIELDS = 26
VOCAB = 100000
D_MODEL = 64
BATCH = 16384
ROWS = N_FIELDS * VOCAB

NC, NS, L = 2, 16, 16
NW = NC * NS
WIN = BATCH // NW
CHUNK = 512
IDX_W = 128
GPC = CHUNK // IDX_W
HPC = WIN // CHUNK
STEPS = N_FIELDS * HPC

_mesh = plsc.VectorSubcoreMesh(core_axis_name="c", subcore_axis_name="s")


def _prep_body(xn_hbm, x1_hbm, vrow):
    wid = lax.axis_index("s") * NC + lax.axis_index("c")
    b0 = wid * WIN
    for f in range(N_FIELDS):
        pltpu.sync_copy(xn_hbm.at[f, pl.ds(b0, WIN)], vrow)
        off = f * VOCAB
        for k in range(WIN // L):
            vrow[pl.ds(k * L, L)] = (vrow[pl.ds(k * L, L)] + off) * 2
        pltpu.sync_copy(vrow, x1_hbm.at[pl.ds(f * BATCH + b0, WIN)])


_prep = pl.kernel(
    _prep_body,
    out_type=jax.ShapeDtypeStruct((N_FIELDS * BATCH,), jnp.int32),
    mesh=_mesh,
    scratch_types=[pltpu.VMEM((WIN,), jnp.int32)],
    compiler_params=pltpu.CompilerParams(use_tc_tiling_on_sc=True),
)


def _gather_body(x1_hbm, tab_hbm, out_hbm, idx0, idx1, rows0, rows1,
                 gs0, gs1, ws0, ws1):
    wid = lax.axis_index("s") * NC + lax.axis_index("c")
    b0 = wid * WIN

    idxs = (idx0, idx1)
    bufs = (rows0, rows1)
    gsems = (gs0, gs1)
    wsems = (ws0, ws1)

    def load_idx(s, b):
        off = pl.multiple_of(
            (s // HPC) * BATCH + b0 + (s % HPC) * CHUNK, IDX_W)
        for q in range(GPC):
            pltpu.sync_copy(x1_hbm.at[pl.ds(off + q * IDX_W, IDX_W)],
                            idxs[b].at[q])

    def fire(b):
        for q in range(GPC):
            pltpu.async_copy(
                tab_hbm.at[idxs[b].at[q]],
                bufs[b].at[pl.ds(q * IDX_W, IDX_W), :],
                gsems[b])

    def wait_gather(b):
        pltpu.make_async_copy(tab_hbm.at[pl.ds(0, CHUNK), :], bufs[b],
                              gsems[b]).wait()

    def put(s, b):
        pltpu.async_copy(
            bufs[b],
            out_hbm.at[s // HPC, pl.ds(b0 + (s % HPC) * CHUNK, CHUNK), :],
            wsems[b])

    def wait_put(b):
        pltpu.make_async_copy(out_hbm.at[0, pl.ds(0, CHUNK), :], bufs[b],
                              wsems[b]).wait()

    load_idx(0, 0)
    fire(0)

    def pair(p, _):
        s0 = 2 * p

        @pl.when(p >= 1)
        def _():
            wait_put(1)
        load_idx(s0 + 1, 1)
        fire(1)

        wait_gather(0)
        put(s0, 0)
        wait_put(0)

        @pl.when(p < STEPS // 2 - 1)
        def _():
            load_idx(s0 + 2, 0)
            fire(0)

        wait_gather(1)
        put(s0 + 1, 1)
        return 0

    lax.fori_loop(0, STEPS // 2, pair, 0)
    wait_put(1)


_gather = pl.kernel(
    _gather_body,
    out_type=jax.ShapeDtypeStruct((N_FIELDS, BATCH, D_MODEL), jnp.float32),
    mesh=_mesh,
    scratch_types=[
        pltpu.VMEM((GPC, IDX_W), jnp.int32),
        pltpu.VMEM((GPC, IDX_W), jnp.int32),
        pltpu.VMEM((CHUNK, D_MODEL), jnp.float32),
        pltpu.VMEM((CHUNK, D_MODEL), jnp.float32),
        pltpu.SemaphoreType.DMA,
        pltpu.SemaphoreType.DMA,
        pltpu.SemaphoreType.DMA,
        pltpu.SemaphoreType.DMA,
    ],
    compiler_params=pltpu.CompilerParams(use_tc_tiling_on_sc=False),
)


@jax.jit
def kernel(x_cat, tables):
    x1 = _prep(x_cat.T)
    tabp = jnp.pad(tables, ((0, 0), (0, 0), (0, 128 - D_MODEL)))
    out = _gather(x1, tabp.reshape(2 * ROWS, D_MODEL))
    return out.transpose(1, 0, 2)
